# baseline (speedup 1.0000x reference)
import jax
import jax.numpy as jnp
from jax import lax
from jax.experimental import pallas as pl
from jax.experimental.pallas import tpu as pltpu
from jax.experimental.pallas import tpu_sc as plsc

B = 1024
D = 128
O = 128
G = 100
K = 4
EPS = 1e-06
LN_EPS = 1e-05

NW = 32
TPW = B // NW
HALF = D // 2


def _prep_body(x_ref, w_ref, b_ref, idx_ref, bb_ref):
    x = x_ref[...]
    mean = jnp.mean(x, axis=-1, keepdims=True)
    var = jnp.mean((x - mean) ** 2, axis=-1, keepdims=True)
    xn = (x - mean) * lax.rsqrt(var + LN_EPS) * w_ref[...] + b_ref[...]
    xc = jnp.clip(xn, -1.0 + EPS, 1.0 - EPS)
    scaled = ((xc + 1.0) / 2.0) * 100.0
    idxf = jnp.floor(scaled)
    t = scaled - idxf
    d_iota = lax.broadcasted_iota(jnp.int32, x.shape, 1)
    idx_ref[...] = d_iota * G + idxf.astype(jnp.int32)
    p1 = t.astype(jnp.bfloat16).astype(jnp.float32)
    p2 = (t * t).astype(jnp.bfloat16).astype(jnp.float32)
    p3 = (t * t * t).astype(jnp.bfloat16).astype(jnp.float32)
    bb_ref[0] = ((1.0 - 3.0 * p1) + 3.0 * p2) - p3
    bb_ref[1] = (3.0 * p1 - 6.0 * p2) + 3.0 * p3
    bb_ref[2] = 3.0 * p2 - 3.0 * p3
    bb_ref[3] = p3


def _prep(x, ln_w, ln_b):
    BT = 256
    return pl.pallas_call(
        _prep_body,
        grid=(B // BT,),
        in_specs=[
            pl.BlockSpec((BT, D), lambda i: (i, 0)),
            pl.BlockSpec((1, D), lambda i: (0, 0)),
            pl.BlockSpec((1, D), lambda i: (0, 0)),
        ],
        out_specs=[
            pl.BlockSpec((BT, D), lambda i: (i, 0)),
            pl.BlockSpec((K, BT, D), lambda i: (0, i, 0)),
        ],
        out_shape=[
            jax.ShapeDtypeStruct((B, D), jnp.int32),
            jax.ShapeDtypeStruct((K, B, D), jnp.float32),
        ],
    )(x, ln_w, ln_b)


def _full16(v):
    return jnp.full((16,), v, jnp.int32)


def _sc_body(pm_hbm, idx_hbm, bb_hbm, out_hbm,
             idxv, bbv, idxs0, idxs1, buf0, buf1, outv, sem0, sem1):
    c = lax.axis_index("c")
    s = lax.axis_index("s")
    wid = s * 2 + c
    base = wid * TPW

    pltpu.sync_copy(idx_hbm.at[pl.ds(base, TPW)], idxv)
    for k in range(K):
        pltpu.sync_copy(bb_hbm.at[pl.ds(k * B * D + base * D, TPW * D)],
                        bbv.at[pl.ds(k * TPW * D, TPW * D)])

    def stage(idxs, i, h):
        for j in range(HALF // 16):
            idxs[pl.ds(j * 16, 16)] = idxv[i, pl.ds(h * HALF + j * 16, 16)]

    def start(idxs, buf, sem):
        pltpu.make_async_copy(pm_hbm.at[idxs], buf, sem).start()

    def wait(idxs, buf, sem):
        pltpu.make_async_copy(pm_hbm.at[idxs], buf, sem).wait()

    def compute(buf, i, h, accs):
        def dbody(jj, accs):
            j0 = jj * 16
            dglob0 = h * HALF + j0
            wks = [bbv[pl.ds(k * (TPW * D) + i * D + dglob0, 16)]
                   for k in range(K)]
            new = list(accs)
            for m in range(16):
                bks = [wks[k][m] for k in range(K)]
                for ci in range(O // 16):
                    a = new[ci]
                    for k in range(K):
                        r = buf[j0 + m, pl.ds(k * O + ci * 16, 16)]
                        a = a + bks[k] * r
                    new[ci] = a
            return tuple(new)
        return lax.fori_loop(0, HALF // 16, dbody, accs)

    zeros = tuple(jnp.zeros((16,), jnp.float32) for _ in range(O // 16))

    stage(idxs0, 0, 0)
    start(idxs0, buf0, sem0)

    def tbody(i, carry):
        stage(idxs1, i, 1)
        start(idxs1, buf1, sem1)
        wait(idxs0, buf0, sem0)
        accs = compute(buf0, i, 0, zeros)

        @pl.when(i + 1 < TPW)
        def _():
            stage(idxs0, i + 1, 0)
            start(idxs0, buf0, sem0)

        wait(idxs1, buf1, sem1)
        accs = compute(buf1, i, 1, accs)
        for ci in range(O // 16):
            outv[i, pl.ds(ci * 16, 16)] = accs[ci]
        return carry

    lax.fori_loop(0, TPW, tbody, 0)
    pltpu.sync_copy(outv, out_hbm.at[pl.ds(base, TPW)])


def _sc_call(pmT, flatidx, bb):
    mesh = plsc.VectorSubcoreMesh(core_axis_name="c", subcore_axis_name="s",
                                  num_cores=2, num_subcores=16)
    return pl.kernel(
        _sc_body,
        out_type=jax.ShapeDtypeStruct((B, O), jnp.float32),
        mesh=mesh,
        scratch_types=[
            pltpu.VMEM((TPW, D), jnp.int32),
            pltpu.VMEM((TPW * D * K + 16,), jnp.float32),
            pltpu.VMEM((HALF,), jnp.int32),
            pltpu.VMEM((HALF,), jnp.int32),
            pltpu.VMEM((HALF, K * O), jnp.float32),
            pltpu.VMEM((HALF, K * O), jnp.float32),
            pltpu.VMEM((TPW, O), jnp.float32),
            pltpu.SemaphoreType.DMA,
            pltpu.SemaphoreType.DMA,
        ],
    )(pmT, flatidx, bb)


def kernel(x, ln_weight, ln_bias, poly_matrix):
    flatidx, bb = _prep(x, ln_weight.reshape(1, D), ln_bias.reshape(1, D))
    pmT = jnp.transpose(poly_matrix, (0, 2, 3, 1)).reshape(D * G, K * O)
    return _sc_call(pmT, flatidx, bb.reshape(-1))


# speedup vs baseline: 2.0670x; 2.0670x over previous
import jax
import jax.numpy as jnp
from jax import lax
from jax.experimental import pallas as pl
from jax.experimental.pallas import tpu as pltpu
from jax.experimental.pallas import tpu_sc as plsc

B = 1024
D = 128
O = 128
G = 100
K = 4
EPS = 1e-06
LN_EPS = 1e-05

NW = 32
TPW = B // NW
HALF = D // 2


def _prep_body(x_ref, w_ref, b_ref, idx_ref, bb_ref):
    x = x_ref[...]
    mean = jnp.mean(x, axis=-1, keepdims=True)
    var = jnp.mean((x - mean) ** 2, axis=-1, keepdims=True)
    xn = (x - mean) * lax.rsqrt(var + LN_EPS) * w_ref[...] + b_ref[...]
    xc = jnp.clip(xn, -1.0 + EPS, 1.0 - EPS)
    scaled = ((xc + 1.0) / 2.0) * 100.0
    idxf = jnp.floor(scaled)
    t = scaled - idxf
    d_iota = lax.broadcasted_iota(jnp.int32, x.shape, 1)
    idx_ref[...] = d_iota * G + idxf.astype(jnp.int32)
    p1 = t.astype(jnp.bfloat16).astype(jnp.float32)
    p2 = (t * t).astype(jnp.bfloat16).astype(jnp.float32)
    p3 = (t * t * t).astype(jnp.bfloat16).astype(jnp.float32)
    bb_ref[0] = ((1.0 - 3.0 * p1) + 3.0 * p2) - p3
    bb_ref[1] = (3.0 * p1 - 6.0 * p2) + 3.0 * p3
    bb_ref[2] = 3.0 * p2 - 3.0 * p3
    bb_ref[3] = p3


def _prep(x, ln_w, ln_b):
    BT = 256
    return pl.pallas_call(
        _prep_body,
        grid=(B // BT,),
        in_specs=[
            pl.BlockSpec((BT, D), lambda i: (i, 0)),
            pl.BlockSpec((1, D), lambda i: (0, 0)),
            pl.BlockSpec((1, D), lambda i: (0, 0)),
        ],
        out_specs=[
            pl.BlockSpec((BT, D), lambda i: (i, 0)),
            pl.BlockSpec((K, BT, D), lambda i: (0, i, 0)),
        ],
        out_shape=[
            jax.ShapeDtypeStruct((B, D), jnp.int32),
            jax.ShapeDtypeStruct((K, B, D), jnp.float32),
        ],
    )(x, ln_w, ln_b)


def _full16(v):
    return jnp.full((16,), v, jnp.int32)


def _sc_body(pm_hbm, idx_hbm, bb_hbm, out_hbm,
             idxv, bbv, idxs0, idxs1, buf0, buf1, outv, sem0, sem1):
    c = lax.axis_index("c")
    s = lax.axis_index("s")
    wid = s * 2 + c
    base = wid * TPW

    pltpu.sync_copy(idx_hbm.at[pl.ds(base, TPW)], idxv)
    for k in range(K):
        pltpu.sync_copy(bb_hbm.at[pl.ds(k * B * D + base * D, TPW * D)],
                        bbv.at[pl.ds(k * TPW * D, TPW * D)])

    def stage(idxs, i, h):
        for j in range(HALF // 16):
            idxs[pl.ds(j * 16, 16)] = idxv[i, pl.ds(h * HALF + j * 16, 16)]

    def start(idxs, buf, sem):
        pltpu.make_async_copy(pm_hbm.at[idxs], buf, sem).start()

    def wait(idxs, buf, sem):
        pltpu.make_async_copy(pm_hbm.at[idxs], buf, sem).wait()

    def compute(buf, i, h, accs):
        def dbody(jj, accs):
            j0 = jj * 2
            new = list(accs)
            for m in range(2):
                dglob = h * HALF + j0 + m
                bks = [bbv[pl.ds(k * (TPW * D) + i * D + dglob, 16)][0]
                       for k in range(K)]
                for ci in range(O // 16):
                    a = new[ci]
                    for k in range(K):
                        r = buf[j0 + m, pl.ds(k * O + ci * 16, 16)]
                        a = a + bks[k] * r
                    new[ci] = a
            return tuple(new)
        return lax.fori_loop(0, HALF // 2, dbody, accs)

    zeros = tuple(jnp.zeros((16,), jnp.float32) for _ in range(O // 16))

    stage(idxs0, 0, 0)
    start(idxs0, buf0, sem0)

    def tbody(i, carry):
        stage(idxs1, i, 1)
        start(idxs1, buf1, sem1)
        wait(idxs0, buf0, sem0)
        accs = compute(buf0, i, 0, zeros)

        @pl.when(i + 1 < TPW)
        def _():
            stage(idxs0, i + 1, 0)
            start(idxs0, buf0, sem0)

        wait(idxs1, buf1, sem1)
        accs = compute(buf1, i, 1, accs)
        for ci in range(O // 16):
            outv[i, pl.ds(ci * 16, 16)] = accs[ci]
        return carry

    lax.fori_loop(0, TPW, tbody, 0)
    pltpu.sync_copy(outv, out_hbm.at[pl.ds(base, TPW)])


def _sc_call(pmT, flatidx, bb):
    mesh = plsc.VectorSubcoreMesh(core_axis_name="c", subcore_axis_name="s",
                                  num_cores=2, num_subcores=16)
    return pl.kernel(
        _sc_body,
        out_type=jax.ShapeDtypeStruct((B, O), jnp.float32),
        mesh=mesh,
        scratch_types=[
            pltpu.VMEM((TPW, D), jnp.int32),
            pltpu.VMEM((TPW * D * K + 16,), jnp.float32),
            pltpu.VMEM((HALF,), jnp.int32),
            pltpu.VMEM((HALF,), jnp.int32),
            pltpu.VMEM((HALF, K * O), jnp.float32),
            pltpu.VMEM((HALF, K * O), jnp.float32),
            pltpu.VMEM((TPW, O), jnp.float32),
            pltpu.SemaphoreType.DMA,
            pltpu.SemaphoreType.DMA,
        ],
    )(pmT, flatidx, bb)


def kernel(x, ln_weight, ln_bias, poly_matrix):
    flatidx, bb = _prep(x, ln_weight.reshape(1, D), ln_bias.reshape(1, D))
    pmT = jnp.transpose(poly_matrix, (0, 2, 3, 1)).reshape(D * G, K * O)
    return _sc_call(pmT, flatidx, bb.reshape(-1))
